# Initial kernel scaffold; baseline (speedup 1.0000x reference)
import jax
import jax.numpy as jnp
from jax.experimental import pallas as pl


def kernel(q, W, b):
    raise NotImplementedError("write your pallas kernel here")


# speedup vs baseline: 2.6844x; 2.6844x over previous
import jax
import jax.numpy as jnp
from jax.experimental import pallas as pl
from jax.experimental.pallas import tpu as pltpu

TOPK_K = 384
C = 768
R = 49
B = 256
BT = 8

_F = jnp.float32


def _erfc_f32(y):
    one = _F(1.0)
    x2 = y * y
    absy = jnp.abs(y)

    p = x2 * _F(7.85386146e-05)
    p = p + _F(-0.000801019371)
    p = p * x2 + _F(0.00518832775)
    p = p * x2 + _F(-0.0268538129)
    p = p * x2 + _F(0.112835854)
    p = p * x2 + _F(-0.37612626)
    p = p * x2 + _F(1.12837911)
    res_lt1 = one - y * p

    z = jnp.exp(-x2)
    rq = z * (one / absy)
    s = one / x2
    pp = s * _F(0.0232682)
    pp = pp + _F(-0.138703942)
    pp = pp * s + _F(0.368742466)
    pp = pp * s + _F(-0.582473278)
    pp = pp * s + _F(0.621000469)
    pp = pp * s + _F(-0.494451523)
    pp = pp * s + _F(0.340488)
    pp = pp * s + _F(-0.274112701)
    pp = pp * s + _F(0.563825965)
    rr = s * _F(-10.477664)
    rr = rr + _F(12.9772)
    rr = rr * s + _F(-7.49551868)
    rr = rr * s + _F(2.92101908)
    rr = rr * s + _F(-1.01526523)
    rr = rr * s + _F(0.42184633)
    rr = rr * s + _F(-0.282076746)
    rr = rr * s + _F(0.564189494)
    val = rq * jnp.where(absy < _F(2.0), pp, rr)
    val = jnp.where(-x2 < _F(-88.7228394), _F(0.0), val)
    res_ge1 = jnp.where(y < _F(0.0), _F(2.0) - val, val)

    return jnp.where(absy < one, res_lt1, res_ge1)


def _select_topk_slots(sm):
    key = jax.lax.bitcast_convert_type(sm, jnp.int32)

    thr = jnp.zeros((B, 1), dtype=jnp.int32)
    for bit in range(30, -1, -1):
        cand = thr | jnp.int32(1 << bit)
        cnt = jnp.sum((key >= cand).astype(jnp.int32), axis=-1, keepdims=True)
        thr = jnp.where(cnt >= TOPK_K, cand, thr)

    gt = key > thr
    eq = key == thr
    n_gt = jnp.sum(gt.astype(jnp.int32), axis=-1, keepdims=True)
    extra = TOPK_K - n_gt

    tri = (jax.lax.broadcasted_iota(jnp.int32, (C, C), 0)
           < jax.lax.broadcasted_iota(jnp.int32, (C, C), 1)
           ).astype(jnp.bfloat16)
    eq_pref = jax.lax.dot_general(
        eq.astype(jnp.bfloat16), tri, (((1,), (0,)), ((), ())),
        preferred_element_type=jnp.float32).astype(jnp.int32)
    sel = gt | (eq & (eq_pref < extra))
    pos = jax.lax.dot_general(
        sel.astype(jnp.bfloat16), tri, (((1,), (0,)), ((), ())),
        preferred_element_type=jnp.float32).astype(jnp.int32)
    return jnp.where(sel, pos, -1)


def _score_body(qt_ref, w_ref, b_ref, slot_ref, sum_ref, max_ref):
    r = pl.program_id(0)
    qr = qt_ref[...].reshape(B, C)

    @pl.when(r == 0)
    def _init():
        sum_ref[...] = qr
        max_ref[...] = qr

    @pl.when(r > 0)
    def _acc():
        sum_ref[...] = sum_ref[...] + qr
        max_ref[...] = jnp.maximum(max_ref[...], qr)

    @pl.when(r == R - 1)
    def _epilogue():
        mean_bf = (sum_ref[...] * _F(0.0204081628)).astype(jnp.bfloat16)
        max_bf = max_ref[...].astype(jnp.bfloat16)
        lhs = jnp.concatenate([mean_bf, max_bf], axis=1)
        x = jax.lax.dot_general(
            lhs, w_ref[...], (((1,), (0,)), ((), ())),
            preferred_element_type=jnp.float32) + b_ref[...]
        g = (x * _F(0.5)) * _erfc_f32(-x * _F(0.707106769))
        mx = jnp.max(g, axis=1, keepdims=True)
        e = jnp.exp(g - mx)
        se = jnp.sum(e, axis=1, keepdims=True)
        sm = e / se
        slot_ref[...] = _select_topk_slots(sm)


def _gather_body(q_ref, slot_ref, out_ref):
    iota_j = jax.lax.broadcasted_iota(jnp.int32, (C, TOPK_K), 1)
    slotT = slot_ref[...].T
    for i in range(BT):
        onehot = jnp.where(slotT[:, i:i + 1] == iota_j,
                           _F(1.0), _F(0.0)).astype(jnp.bfloat16)
        qi = q_ref[i]
        hi = qi.astype(jnp.bfloat16)
        lo = (qi - hi.astype(jnp.float32)).astype(jnp.bfloat16)
        out_ref[i] = (
            jax.lax.dot_general(hi, onehot, (((1,), (0,)), ((), ())),
                                preferred_element_type=jnp.float32)
            + jax.lax.dot_general(lo, onehot, (((1,), (0,)), ((), ())),
                                  preferred_element_type=jnp.float32))


@jax.jit
def kernel(q, W, b):
    b2 = b.reshape(1, C)
    qt = jnp.transpose(q, (1, 0, 2))
    slot = pl.pallas_call(
        _score_body,
        grid=(R,),
        in_specs=[
            pl.BlockSpec((1, B, C), lambda r: (r, 0, 0)),
            pl.BlockSpec((2 * C, C), lambda r: (0, 0)),
            pl.BlockSpec((1, C), lambda r: (0, 0)),
        ],
        out_specs=pl.BlockSpec((B, C), lambda r: (0, 0)),
        out_shape=jax.ShapeDtypeStruct((B, C), jnp.int32),
        scratch_shapes=[
            pltpu.VMEM((B, C), jnp.float32),
            pltpu.VMEM((B, C), jnp.float32),
        ],
    )(qt, W, b2)

    return pl.pallas_call(
        _gather_body,
        grid=(B // BT,),
        in_specs=[
            pl.BlockSpec((BT, R, C), lambda i: (i, 0, 0)),
            pl.BlockSpec((BT, C), lambda i: (i, 0)),
        ],
        out_specs=pl.BlockSpec((BT, R, TOPK_K), lambda i: (i, 0, 0)),
        out_shape=jax.ShapeDtypeStruct((B, R, TOPK_K), jnp.float32),
    )(q, slot)
